# baseline (speedup 1.0000x reference)
import functools

import jax
import jax.numpy as jnp
from jax import lax
from jax.experimental import pallas as pl
from jax.experimental.pallas import tpu as pltpu
from jax.experimental.pallas import tpu_sc as plsc

NC = 2
NS = 16
NW = NC * NS
LANES = 16

B = 16384
K = 128
B_PER_W = B // NW
CHUNK = 64
N_CHUNKS = B_PER_W // CHUNK


def _make_kernel():
  mesh = plsc.VectorSubcoreMesh(
      core_axis_name="c", subcore_axis_name="s",
      num_cores=NC, num_subcores=NS)

  @functools.partial(
      pl.kernel,
      out_type=jax.ShapeDtypeStruct((B,), jnp.float32),
      mesh=mesh,
      compiler_params=pltpu.CompilerParams(needs_layout_passes=False),
      scratch_types=[
          pltpu.VMEM((N_CHUNKS, CHUNK), jnp.int32),
          pltpu.VMEM((N_CHUNKS, CHUNK), jnp.int32),
          pltpu.VMEM((CHUNK, K), jnp.float32),
          pltpu.VMEM((CHUNK, K), jnp.float32),
          pltpu.VMEM((CHUNK, K), jnp.float32),
          pltpu.VMEM((CHUNK, K), jnp.float32),
          pltpu.VMEM((CHUNK, K), jnp.float32),
          pltpu.VMEM((CHUNK, K), jnp.float32),
          pltpu.VMEM((B_PER_W,), jnp.float32),
          pltpu.SemaphoreType.DMA,
          pltpu.SemaphoreType.DMA,
          pltpu.SemaphoreType.DMA,
      ],
  )
  def cf_kernel(uidx_hbm, iidx_hbm, utab_hbm, itab_hbm, out_hbm,
                uidx_v, iidx_v, urows0, irows0, urows1, irows1,
                urows2, irows2, out_v, sem0, sem1, sem2):
    wid = lax.axis_index("s") * NC + lax.axis_index("c")
    base = wid * B_PER_W

    cu0 = pltpu.async_copy(uidx_hbm.at[wid], uidx_v, sem0)
    ci0 = pltpu.async_copy(iidx_hbm.at[wid], iidx_v, sem1)
    cu0.wait()
    ci0.wait()

    lane_iota = lax.iota(jnp.int32, LANES)
    bufs = ((urows0, irows0, sem0), (urows1, irows1, sem1),
            (urows2, irows2, sem2))
    NBUF = len(bufs)

    def issue(c):
      urows, irows, sem = bufs[c % NBUF]
      cu = pltpu.async_copy(utab_hbm.at[uidx_v.at[c]], urows, sem)
      ci = pltpu.async_copy(itab_hbm.at[iidx_v.at[c]], irows, sem)
      return cu, ci

    pending = [issue(c) for c in range(NBUF - 1)]
    for c in range(N_CHUNKS):
      cu, ci = pending.pop(0)
      cu.wait()
      ci.wait()
      urows, irows, _ = bufs[c % NBUF]
      if c + NBUF - 1 < N_CHUNKS:
        pending.append(issue(c + NBUF - 1))

      def row_body(r, acc16, c=c, urows=urows, irows=irows):
        part = urows[r, pl.ds(0, LANES)] * irows[r, pl.ds(0, LANES)]
        for j in range(1, K // LANES):
          part = part + (urows[r, pl.ds(j * LANES, LANES)]
                         * irows[r, pl.ds(j * LANES, LANES)])
        s = jnp.sum(part)
        acc16 = jnp.where(lane_iota == (r & (LANES - 1)), s, acc16)

        @pl.when((r & (LANES - 1)) == LANES - 1)
        def _():
          out_v[pl.ds(c * CHUNK + r - (LANES - 1), LANES)] = acc16

        return acc16

      lax.fori_loop(0, CHUNK, row_body,
                    jnp.zeros((LANES,), jnp.float32), unroll=4)

    pltpu.sync_copy(out_v, out_hbm.at[pl.ds(base, B_PER_W)])

  return cf_kernel


_cf_kernel = _make_kernel()


@jax.jit
def kernel(user_input, item_input, user_table, item_table):
  uidx = user_input.astype(jnp.int32).reshape(NW, N_CHUNKS, CHUNK)
  iidx = item_input.astype(jnp.int32).reshape(NW, N_CHUNKS, CHUNK)
  out = _cf_kernel(uidx, iidx, user_table, item_table)
  return out.reshape(B, 1)


# speedup vs baseline: 1.0459x; 1.0459x over previous
import functools

import jax
import jax.numpy as jnp
from jax import lax
from jax.experimental import pallas as pl
from jax.experimental.pallas import tpu as pltpu
from jax.experimental.pallas import tpu_sc as plsc

NC = 2
NS = 16
NW = NC * NS
LANES = 16

B = 16384
K = 128
B_PER_W = B // NW
CHUNK = 128
N_CHUNKS = B_PER_W // CHUNK


def _make_kernel():
  mesh = plsc.VectorSubcoreMesh(
      core_axis_name="c", subcore_axis_name="s",
      num_cores=NC, num_subcores=NS)

  @functools.partial(
      pl.kernel,
      out_type=jax.ShapeDtypeStruct((B,), jnp.float32),
      mesh=mesh,
      compiler_params=pltpu.CompilerParams(needs_layout_passes=False),
      scratch_types=[
          pltpu.VMEM((N_CHUNKS, CHUNK), jnp.int32),
          pltpu.VMEM((N_CHUNKS, CHUNK), jnp.int32),
          pltpu.VMEM((CHUNK, K), jnp.float32),
          pltpu.VMEM((CHUNK, K), jnp.float32),
          pltpu.VMEM((CHUNK, K), jnp.float32),
          pltpu.VMEM((CHUNK, K), jnp.float32),
          pltpu.VMEM((CHUNK, K), jnp.float32),
          pltpu.VMEM((CHUNK, K), jnp.float32),
          pltpu.VMEM((B_PER_W,), jnp.float32),
          pltpu.SemaphoreType.DMA,
          pltpu.SemaphoreType.DMA,
          pltpu.SemaphoreType.DMA,
      ],
  )
  def cf_kernel(uidx_hbm, iidx_hbm, utab_hbm, itab_hbm, out_hbm,
                uidx_v, iidx_v, urows0, irows0, urows1, irows1,
                urows2, irows2, out_v, sem0, sem1, sem2):
    wid = lax.axis_index("s") * NC + lax.axis_index("c")
    base = wid * B_PER_W

    cu0 = pltpu.async_copy(uidx_hbm.at[wid], uidx_v, sem0)
    ci0 = pltpu.async_copy(iidx_hbm.at[wid], iidx_v, sem1)
    cu0.wait()
    ci0.wait()

    lane_iota = lax.iota(jnp.int32, LANES)
    bufs = ((urows0, irows0, sem0), (urows1, irows1, sem1),
            (urows2, irows2, sem2))
    NBUF = len(bufs)

    def issue(c):
      urows, irows, sem = bufs[c % NBUF]
      cu = pltpu.async_copy(utab_hbm.at[uidx_v.at[c]], urows, sem)
      ci = pltpu.async_copy(itab_hbm.at[iidx_v.at[c]], irows, sem)
      return cu, ci

    pending = [issue(c) for c in range(NBUF - 1)]
    for c in range(N_CHUNKS):
      cu, ci = pending.pop(0)
      cu.wait()
      ci.wait()
      urows, irows, _ = bufs[c % NBUF]
      if c + NBUF - 1 < N_CHUNKS:
        pending.append(issue(c + NBUF - 1))

      def row_body(r, acc16, c=c, urows=urows, irows=irows):
        part = urows[r, pl.ds(0, LANES)] * irows[r, pl.ds(0, LANES)]
        for j in range(1, K // LANES):
          part = part + (urows[r, pl.ds(j * LANES, LANES)]
                         * irows[r, pl.ds(j * LANES, LANES)])
        s = jnp.sum(part)
        acc16 = jnp.where(lane_iota == (r & (LANES - 1)), s, acc16)

        @pl.when((r & (LANES - 1)) == LANES - 1)
        def _():
          out_v[pl.ds(c * CHUNK + r - (LANES - 1), LANES)] = acc16

        return acc16

      lax.fori_loop(0, CHUNK, row_body,
                    jnp.zeros((LANES,), jnp.float32), unroll=4)

    pltpu.sync_copy(out_v, out_hbm.at[pl.ds(base, B_PER_W)])

  return cf_kernel


_cf_kernel = _make_kernel()


@jax.jit
def kernel(user_input, item_input, user_table, item_table):
  uidx = user_input.astype(jnp.int32).reshape(NW, N_CHUNKS, CHUNK)
  iidx = item_input.astype(jnp.int32).reshape(NW, N_CHUNKS, CHUNK)
  out = _cf_kernel(uidx, iidx, user_table, item_table)
  return out.reshape(B, 1)
